# Initial kernel scaffold; baseline (speedup 1.0000x reference)
import jax
import jax.numpy as jnp
from jax.experimental import pallas as pl


def kernel(indices, word_embeddings):
    raise NotImplementedError("write your pallas kernel here")


# speedup vs baseline: 1.1030x; 1.1030x over previous
import jax
import jax.numpy as jnp
from jax import lax
from jax.experimental import pallas as pl
from jax.experimental.pallas import tpu as pltpu
from jax.experimental.pallas import tpu_sc as plsc

VOCAB = 100
HIDDEN = 1024
BATCH = 4096
SEQ = 20

NC, NS, L = 2, 16, 16
NW = NC * NS
NROWS = BATCH * SEQ
BPW = NROWS // NW
C = 40
NCH = BPW // C

_mesh = plsc.VectorSubcoreMesh(core_axis_name="c", subcore_axis_name="s")


@jax.jit
def _sc_gather(table, idx):
    @pl.kernel(
        out_type=jax.ShapeDtypeStruct((NROWS, HIDDEN), jnp.float32),
        mesh=_mesh,
        scratch_types=[
            pltpu.VMEM((NCH, C), jnp.int32),
            pltpu.VMEM((C, HIDDEN), jnp.float32),
            pltpu.VMEM((C, HIDDEN), jnp.float32),
            pltpu.SemaphoreType.DMA,
            pltpu.SemaphoreType.DMA,
        ],
    )
    def k(table_hbm, idx_hbm, out_hbm, idx_v, buf0, buf1, sem0, sem1):
        wid = lax.axis_index("s") * NC + lax.axis_index("c")
        base = wid * BPW
        pltpu.sync_copy(idx_hbm.at[wid], idx_v)
        bufs = (buf0, buf1)
        sems = (sem0, sem1)
        pltpu.async_copy(table_hbm.at[idx_v.at[0]], buf0, sem0)

        def one_chunk(g, b):
            @pl.when(g + 1 < NCH)
            def _():
                pltpu.async_copy(
                    table_hbm.at[idx_v.at[g + 1]], bufs[1 - b], sems[1 - b]
                )
            pltpu.make_async_copy(
                table_hbm.at[idx_v.at[g]], bufs[b], sems[b]
            ).wait()
            pltpu.sync_copy(bufs[b], out_hbm.at[pl.ds(base + g * C, C)])

        def outer(i, carry):
            one_chunk(i * 2, 0)
            one_chunk(i * 2 + 1, 1)
            return carry

        lax.fori_loop(0, NCH // 2, outer, 0)

    return k(table, idx)


def kernel(indices, word_embeddings):
    idx = indices.reshape(NW, NCH, C)
    out = _sc_gather(word_embeddings, idx)
    return out.reshape(BATCH, SEQ, HIDDEN)
